# baseline (speedup 1.0000x reference)
import functools

import jax
import jax.numpy as jnp
from jax import lax
from jax.experimental import pallas as pl
from jax.experimental.pallas import tpu as pltpu
from jax.experimental.pallas import tpu_sc as plsc

CONF_THRES = 0.25
NEG_INF = float("-inf")
BIG = 1 << 30

B, N, D = 8, 5000, 85
NCLS = D - 5
NPAD = 5120
TILE0 = 1280
TILE3 = 1160
PAD = 1280
CH = 640
NCHUNK = PAD // CH
CG = CH // 16
LAST3 = TILE3 - CH
NLANES = 8

_PERM_DN = lax.GatherDimensionNumbers(
    offset_dims=(), collapsed_slice_dims=(0,), start_index_map=(0,))


def _lane_perm(v, idx):
    return lax.gather(v, idx[:, None], _PERM_DN, (1,),
                      mode=lax.GatherScatterMode.PROMISE_IN_BOUNDS)


def _sc_body(pred_hbm, cc_hbm, cp_hbm, mask_hbm, scores_hbm,
             slab, conf_buf, valid_buf, mask_buf, scores_buf,
             cc_buf, cp_buf, v16f, v16i, counts_sh, top5v_sh, top5i_sh):
    c = lax.axis_index("c")
    s = lax.axis_index("s")
    wid = c * 16 + s
    b = wid // 4
    j = wid % 4
    b_loc = s // 4
    nbase = j * TILE0
    n = jnp.where(j == 3, TILE3, TILE0)
    iota16 = lax.iota(jnp.int32, 16)

    def run_chunk(k, cnt):
        def group(g, cnt):
            lr = g * 16 + iota16
            glr = k * CH + lr

            def fld(f):
                return plsc.load_gather(
                    slab, [lr, jnp.full((16,), f, jnp.int32)])

            v0 = fld(0)
            v1 = fld(1)
            v2 = fld(2)
            v3 = fld(3)
            conf_v = fld(4)

            ccv = jnp.zeros((16,), jnp.float32)
            cpv = jnp.zeros((16,), jnp.int32)
            for i in range(16):
                r = g * 16 + i
                va = slab[r, pl.ds(0, 16)]
                vb = slab[r, pl.ds(16, 16)]
                vc = slab[r, pl.ds(32, 16)]
                vd = slab[r, pl.ds(48, 16)]
                ve = slab[r, pl.ds(64, 16)]
                vf = slab[r, pl.ds(69, 16)]
                va = jnp.where(iota16 >= 5, va, NEG_INF)
                m = jnp.maximum(jnp.maximum(jnp.maximum(va, vb),
                                            jnp.maximum(vc, vd)),
                                jnp.maximum(ve, vf))
                for sh in (1, 2, 4, 8):
                    m = jnp.maximum(m, _lane_perm(m, iota16 ^ sh))
                mval = m
                cand = jnp.minimum(
                    jnp.minimum(
                        jnp.minimum(
                            jnp.where(va == mval, iota16 - 5, BIG),
                            jnp.where(vb == mval, iota16 + 11, BIG)),
                        jnp.minimum(
                            jnp.where(vc == mval, iota16 + 27, BIG),
                            jnp.where(vd == mval, iota16 + 43, BIG))),
                    jnp.minimum(
                        jnp.where(ve == mval, iota16 + 59, BIG),
                        jnp.where(vf == mval, iota16 + 64, BIG)))
                for sh in (1, 2, 4, 8):
                    cand = jnp.minimum(cand, _lane_perm(cand, iota16 ^ sh))
                here = iota16 == i
                ccv = jnp.where(here, mval, ccv)
                cpv = jnp.where(here, cand, cpv)

            real = glr < n
            cmaskv = (conf_v >= CONF_THRES) & real
            validv = (v2 > v0) & (v3 > v1)
            finalv = cmaskv & validv
            cnt = cnt + jnp.where(cmaskv, jnp.int32(1), jnp.int32(0))

            sl = pl.ds(k * CH + g * 16, 16)
            mask_buf[sl] = jnp.where(finalv, jnp.int32(1), jnp.int32(0))
            scores_buf[sl] = jnp.where(finalv, conf_v, NEG_INF)
            conf_buf[sl] = jnp.where(real, conf_v, NEG_INF)
            valid_buf[sl] = jnp.where(validv, jnp.int32(1), jnp.int32(0))
            cc_buf[sl] = ccv
            cp_buf[sl] = cpv.astype(jnp.float32)
            return cnt

        return plsc.parallel_loop(0, CG, carry=cnt, unroll=1)(group)

    cnt16 = jnp.zeros((16,), jnp.int32)
    for k in range(NCHUNK):
        row0 = nbase + k * CH
        if k < NCHUNK - 1:
            pltpu.sync_copy(pred_hbm.at[b, pl.ds(row0, CH)], slab)
        else:
            @pl.when(j < 3)
            def _full_last():
                pltpu.sync_copy(pred_hbm.at[b, pl.ds(row0, CH)], slab)

            @pl.when(j == 3)
            def _part_last():
                pltpu.sync_copy(pred_hbm.at[b, pl.ds(row0, LAST3)],
                                slab.at[pl.ds(0, LAST3)])

        cnt16 = run_chunk(k, cnt16)

    obase = b * NPAD + nbase
    pltpu.sync_copy(cc_buf, cc_hbm.at[pl.ds(obase, PAD)])
    pltpu.sync_copy(cp_buf, cp_hbm.at[pl.ds(obase, PAD)])
    pltpu.sync_copy(mask_buf, mask_hbm.at[pl.ds(obase, PAD)])
    pltpu.sync_copy(scores_buf, scores_hbm.at[pl.ds(obase, PAD)])

    total = jnp.sum(cnt16)
    v16i[...] = jnp.where(iota16 == 0, total, jnp.int32(0))
    pltpu.sync_copy(v16i, counts_sh.at[s])
    plsc.subcore_barrier()
    s0 = b_loc * 4
    above = jnp.int32(0)
    for m in range(4):
        pltpu.sync_copy(counts_sh.at[s0 + m], v16i)
        above = above + jnp.sum(v16i[...])
    need_fb = above == 0

    NG = PAD // 16
    v16f[...] = jnp.full((16,), NEG_INF, jnp.float32)
    pltpu.sync_copy(v16f, top5v_sh.at[s])
    v16i[...] = jnp.full((16,), BIG, jnp.int32)
    pltpu.sync_copy(v16i, top5i_sh.at[s])

    @pl.when(need_fb)
    def _local_top5():
        t5v = jnp.full((16,), NEG_INF, jnp.float32)
        t5i = jnp.full((16,), BIG, jnp.int32)
        sel = []
        for k in range(5):
            m16 = lax.fori_loop(
                0, NG,
                lambda g, m: jnp.maximum(m, conf_buf[pl.ds(g * 16, 16)]),
                jnp.full((16,), NEG_INF, jnp.float32))
            mval = jnp.max(m16)

            def first_idx(g, cur):
                v = conf_buf[pl.ds(g * 16, 16)]
                cand = jnp.min(jnp.where(v == mval, g * 16 + iota16, BIG))
                return jnp.minimum(cur, cand)

            idx = lax.fori_loop(0, NG, first_idx, jnp.int32(BIG))
            t5v = jnp.where(iota16 == k, mval, t5v)
            t5i = jnp.where(iota16 == k, nbase + idx, t5i)
            sel.append((idx, mval))
            gsel = idx // 16
            hole = pl.ds(gsel * 16, 16)
            conf_buf[hole] = jnp.where(gsel * 16 + iota16 == idx, NEG_INF,
                                       conf_buf[hole])
        for idx, mval in reversed(sel):
            gsel = idx // 16
            hole = pl.ds(gsel * 16, 16)
            conf_buf[hole] = jnp.where(gsel * 16 + iota16 == idx, mval,
                                       conf_buf[hole])
        v16f[...] = t5v
        pltpu.sync_copy(v16f, top5v_sh.at[s])
        v16i[...] = t5i
        pltpu.sync_copy(v16i, top5i_sh.at[s])

    plsc.subcore_barrier()

    @pl.when(need_fb)
    def _apply_fb():
        vs, ix = [], []
        for m in range(4):
            pltpu.sync_copy(top5v_sh.at[s0 + m], v16f)
            vs.append(v16f[...])
            pltpu.sync_copy(top5i_sh.at[s0 + m], v16i)
            ix.append(v16i[...])
        sel = []
        for k in range(5):
            mval = jnp.max(jnp.maximum(jnp.maximum(vs[0], vs[1]),
                                       jnp.maximum(vs[2], vs[3])))
            cand = jnp.int32(BIG)
            for m in range(4):
                cand = jnp.minimum(
                    cand, jnp.min(jnp.where(vs[m] == mval, ix[m], BIG)))
            for m in range(4):
                hit = (vs[m] == mval) & (ix[m] == cand)
                vs[m] = jnp.where(hit, NEG_INF, vs[m])
            sel.append(cand)

        def rebuild(g, _):
            lr = g * 16 + iota16
            blr = nbase + lr
            inb = ((blr == sel[0]) | (blr == sel[1]) | (blr == sel[2])
                   | (blr == sel[3]) | (blr == sel[4]))
            sl = pl.ds(g * 16, 16)
            fin = inb & (valid_buf[sl] != 0) & (lr < n)
            mask_buf[sl] = jnp.where(fin, jnp.int32(1), jnp.int32(0))
            scores_buf[sl] = jnp.where(fin, conf_buf[sl], NEG_INF)
            return 0

        lax.fori_loop(0, NG, rebuild, 0)
        pltpu.sync_copy(mask_buf, mask_hbm.at[pl.ds(obase, PAD)])
        pltpu.sync_copy(scores_buf, scores_hbm.at[pl.ds(obase, PAD)])


_sc_call = pl.kernel(
    _sc_body,
    out_type=(
        jax.ShapeDtypeStruct((B * NPAD,), jnp.float32),
        jax.ShapeDtypeStruct((B * NPAD,), jnp.float32),
        jax.ShapeDtypeStruct((B * NPAD,), jnp.int32),
        jax.ShapeDtypeStruct((B * NPAD,), jnp.float32),
    ),
    mesh=plsc.VectorSubcoreMesh(core_axis_name="c", subcore_axis_name="s",
                                num_cores=2, num_subcores=16),
    compiler_params=pltpu.CompilerParams(needs_layout_passes=False),
    scratch_types=[
        pltpu.VMEM((CH, D), jnp.float32),
        pltpu.VMEM((PAD,), jnp.float32),
        pltpu.VMEM((PAD,), jnp.int32),
        pltpu.VMEM((PAD,), jnp.int32),
        pltpu.VMEM((PAD,), jnp.float32),
        pltpu.VMEM((PAD,), jnp.float32),
        pltpu.VMEM((PAD,), jnp.float32),
        pltpu.VMEM((16,), jnp.float32),
        pltpu.VMEM((16,), jnp.int32),
        pltpu.VMEM_SHARED((16, 16), jnp.int32),
        pltpu.VMEM_SHARED((16, 16), jnp.float32),
        pltpu.VMEM_SHARED((16, 16), jnp.int32),
    ],
)


@jax.jit
def kernel(prediction):
    cc, cp, mask_i32, scores = _sc_call(prediction)
    cc = cc.reshape(B, NPAD)[:, :N, None]
    cp = cp.reshape(B, NPAD)[:, :N, None]
    det = jnp.concatenate([prediction[:, :, :5], cc, cp], axis=-1)
    mask = mask_i32.reshape(B, NPAD)[:, :N] != 0
    return det, mask, scores.reshape(B, NPAD)[:, :N]


# speedup vs baseline: 1.2635x; 1.2635x over previous
import functools

import jax
import jax.numpy as jnp
from jax import lax
from jax.experimental import pallas as pl
from jax.experimental.pallas import tpu as pltpu
from jax.experimental.pallas import tpu_sc as plsc

CONF_THRES = 0.25
NEG_INF = float("-inf")
BIG = 1 << 30

B, N, D = 8, 5000, 85
NCLS = D - 5
NPAD = 5120
TILE0 = 1280
TILE3 = 1160
PAD = 1280
CH = 640
NCHUNK = PAD // CH
CG = CH // 16
LAST3 = TILE3 - CH
NLANES = 8


def _sc_body(pred_hbm, cc_hbm, cp_hbm, mask_hbm, scores_hbm,
             slab, conf_buf, valid_buf, mask_buf, scores_buf,
             cc_buf, cp_buf, v16f, v16i, counts_sh, top5v_sh, top5i_sh):
    c = lax.axis_index("c")
    s = lax.axis_index("s")
    wid = c * 16 + s
    b = wid // 4
    j = wid % 4
    b_loc = s // 4
    nbase = j * TILE0
    n = jnp.where(j == 3, TILE3, TILE0)
    iota16 = lax.iota(jnp.int32, 16)

    def run_chunk(k, cnt):
        def group(g, cnt):
            lr = g * 16 + iota16
            glr = k * CH + lr

            def fld(f):
                return plsc.load_gather(
                    slab, [lr, jnp.full((16,), f, jnp.int32)])

            v0 = fld(0)
            v1 = fld(1)
            v2 = fld(2)
            v3 = fld(3)
            conf_v = fld(4)

            ccv = jnp.zeros((16,), jnp.float32)
            cpv = jnp.zeros((16,), jnp.int32)
            for i in range(16):
                r = g * 16 + i
                va = slab[r, pl.ds(0, 16)]
                vb = slab[r, pl.ds(16, 16)]
                vc = slab[r, pl.ds(32, 16)]
                vd = slab[r, pl.ds(48, 16)]
                ve = slab[r, pl.ds(64, 16)]
                vf = slab[r, pl.ds(69, 16)]
                va = jnp.where(iota16 >= 5, va, NEG_INF)
                m = jnp.maximum(jnp.maximum(jnp.maximum(va, vb),
                                            jnp.maximum(vc, vd)),
                                jnp.maximum(ve, vf))
                mval = jnp.max(m)
                cand = jnp.minimum(
                    jnp.minimum(
                        jnp.minimum(
                            jnp.where(va == mval, iota16 - 5, BIG),
                            jnp.where(vb == mval, iota16 + 11, BIG)),
                        jnp.minimum(
                            jnp.where(vc == mval, iota16 + 27, BIG),
                            jnp.where(vd == mval, iota16 + 43, BIG))),
                    jnp.minimum(
                        jnp.where(ve == mval, iota16 + 59, BIG),
                        jnp.where(vf == mval, iota16 + 64, BIG)))
                cidx = jnp.min(cand)
                here = iota16 == i
                ccv = jnp.where(here, mval, ccv)
                cpv = jnp.where(here, cidx, cpv)

            real = glr < n
            cmaskv = (conf_v >= CONF_THRES) & real
            validv = (v2 > v0) & (v3 > v1)
            finalv = cmaskv & validv
            cnt = cnt + jnp.where(cmaskv, jnp.int32(1), jnp.int32(0))

            sl = pl.ds(k * CH + g * 16, 16)
            mask_buf[sl] = jnp.where(finalv, jnp.int32(1), jnp.int32(0))
            scores_buf[sl] = jnp.where(finalv, conf_v, NEG_INF)
            conf_buf[sl] = jnp.where(real, conf_v, NEG_INF)
            valid_buf[sl] = jnp.where(validv, jnp.int32(1), jnp.int32(0))
            cc_buf[sl] = ccv
            cp_buf[sl] = cpv.astype(jnp.float32)
            return cnt

        return plsc.parallel_loop(0, CG, carry=cnt, unroll=1)(group)

    cnt16 = jnp.zeros((16,), jnp.int32)
    for k in range(NCHUNK):
        row0 = nbase + k * CH
        if k < NCHUNK - 1:
            pltpu.sync_copy(pred_hbm.at[b, pl.ds(row0, CH)], slab)
        else:
            @pl.when(j < 3)
            def _full_last():
                pltpu.sync_copy(pred_hbm.at[b, pl.ds(row0, CH)], slab)

            @pl.when(j == 3)
            def _part_last():
                pltpu.sync_copy(pred_hbm.at[b, pl.ds(row0, LAST3)],
                                slab.at[pl.ds(0, LAST3)])

        cnt16 = run_chunk(k, cnt16)

    obase = b * NPAD + nbase
    pltpu.sync_copy(cc_buf, cc_hbm.at[pl.ds(obase, PAD)])
    pltpu.sync_copy(cp_buf, cp_hbm.at[pl.ds(obase, PAD)])
    pltpu.sync_copy(mask_buf, mask_hbm.at[pl.ds(obase, PAD)])
    pltpu.sync_copy(scores_buf, scores_hbm.at[pl.ds(obase, PAD)])

    total = jnp.sum(cnt16)
    v16i[...] = jnp.where(iota16 == 0, total, jnp.int32(0))
    pltpu.sync_copy(v16i, counts_sh.at[s])
    plsc.subcore_barrier()
    s0 = b_loc * 4
    above = jnp.int32(0)
    for m in range(4):
        pltpu.sync_copy(counts_sh.at[s0 + m], v16i)
        above = above + jnp.sum(v16i[...])
    need_fb = above == 0

    NG = PAD // 16
    v16f[...] = jnp.full((16,), NEG_INF, jnp.float32)
    pltpu.sync_copy(v16f, top5v_sh.at[s])
    v16i[...] = jnp.full((16,), BIG, jnp.int32)
    pltpu.sync_copy(v16i, top5i_sh.at[s])

    @pl.when(need_fb)
    def _local_top5():
        t5v = jnp.full((16,), NEG_INF, jnp.float32)
        t5i = jnp.full((16,), BIG, jnp.int32)
        sel = []
        for k in range(5):
            m16 = lax.fori_loop(
                0, NG,
                lambda g, m: jnp.maximum(m, conf_buf[pl.ds(g * 16, 16)]),
                jnp.full((16,), NEG_INF, jnp.float32))
            mval = jnp.max(m16)

            def first_idx(g, cur):
                v = conf_buf[pl.ds(g * 16, 16)]
                cand = jnp.min(jnp.where(v == mval, g * 16 + iota16, BIG))
                return jnp.minimum(cur, cand)

            idx = lax.fori_loop(0, NG, first_idx, jnp.int32(BIG))
            t5v = jnp.where(iota16 == k, mval, t5v)
            t5i = jnp.where(iota16 == k, nbase + idx, t5i)
            sel.append((idx, mval))
            gsel = idx // 16
            hole = pl.ds(gsel * 16, 16)
            conf_buf[hole] = jnp.where(gsel * 16 + iota16 == idx, NEG_INF,
                                       conf_buf[hole])
        for idx, mval in reversed(sel):
            gsel = idx // 16
            hole = pl.ds(gsel * 16, 16)
            conf_buf[hole] = jnp.where(gsel * 16 + iota16 == idx, mval,
                                       conf_buf[hole])
        v16f[...] = t5v
        pltpu.sync_copy(v16f, top5v_sh.at[s])
        v16i[...] = t5i
        pltpu.sync_copy(v16i, top5i_sh.at[s])

    plsc.subcore_barrier()

    @pl.when(need_fb)
    def _apply_fb():
        vs, ix = [], []
        for m in range(4):
            pltpu.sync_copy(top5v_sh.at[s0 + m], v16f)
            vs.append(v16f[...])
            pltpu.sync_copy(top5i_sh.at[s0 + m], v16i)
            ix.append(v16i[...])
        sel = []
        for k in range(5):
            mval = jnp.max(jnp.maximum(jnp.maximum(vs[0], vs[1]),
                                       jnp.maximum(vs[2], vs[3])))
            cand = jnp.int32(BIG)
            for m in range(4):
                cand = jnp.minimum(
                    cand, jnp.min(jnp.where(vs[m] == mval, ix[m], BIG)))
            for m in range(4):
                hit = (vs[m] == mval) & (ix[m] == cand)
                vs[m] = jnp.where(hit, NEG_INF, vs[m])
            sel.append(cand)

        def rebuild(g, _):
            lr = g * 16 + iota16
            blr = nbase + lr
            inb = ((blr == sel[0]) | (blr == sel[1]) | (blr == sel[2])
                   | (blr == sel[3]) | (blr == sel[4]))
            sl = pl.ds(g * 16, 16)
            fin = inb & (valid_buf[sl] != 0) & (lr < n)
            mask_buf[sl] = jnp.where(fin, jnp.int32(1), jnp.int32(0))
            scores_buf[sl] = jnp.where(fin, conf_buf[sl], NEG_INF)
            return 0

        lax.fori_loop(0, NG, rebuild, 0)
        pltpu.sync_copy(mask_buf, mask_hbm.at[pl.ds(obase, PAD)])
        pltpu.sync_copy(scores_buf, scores_hbm.at[pl.ds(obase, PAD)])


_sc_call = pl.kernel(
    _sc_body,
    out_type=(
        jax.ShapeDtypeStruct((B * NPAD,), jnp.float32),
        jax.ShapeDtypeStruct((B * NPAD,), jnp.float32),
        jax.ShapeDtypeStruct((B * NPAD,), jnp.int32),
        jax.ShapeDtypeStruct((B * NPAD,), jnp.float32),
    ),
    mesh=plsc.VectorSubcoreMesh(core_axis_name="c", subcore_axis_name="s",
                                num_cores=2, num_subcores=16),
    compiler_params=pltpu.CompilerParams(needs_layout_passes=False),
    scratch_types=[
        pltpu.VMEM((CH, D), jnp.float32),
        pltpu.VMEM((PAD,), jnp.float32),
        pltpu.VMEM((PAD,), jnp.int32),
        pltpu.VMEM((PAD,), jnp.int32),
        pltpu.VMEM((PAD,), jnp.float32),
        pltpu.VMEM((PAD,), jnp.float32),
        pltpu.VMEM((PAD,), jnp.float32),
        pltpu.VMEM((16,), jnp.float32),
        pltpu.VMEM((16,), jnp.int32),
        pltpu.VMEM_SHARED((16, 16), jnp.int32),
        pltpu.VMEM_SHARED((16, 16), jnp.float32),
        pltpu.VMEM_SHARED((16, 16), jnp.int32),
    ],
)


@jax.jit
def kernel(prediction):
    cc, cp, mask_i32, scores = _sc_call(prediction)
    cc = cc.reshape(B, NPAD)[:, :N, None]
    cp = cp.reshape(B, NPAD)[:, :N, None]
    det = jnp.concatenate([prediction[:, :, :5], cc, cp], axis=-1)
    mask = mask_i32.reshape(B, NPAD)[:, :N] != 0
    return det, mask, scores.reshape(B, NPAD)[:, :N]


# speedup vs baseline: 1.6886x; 1.3365x over previous
import jax
import jax.numpy as jnp
from jax import lax
from jax.experimental import pallas as pl
from jax.experimental.pallas import tpu as pltpu
from jax.experimental.pallas import tpu_sc as plsc

CONF_THRES = 0.25
NEG_INF = float("-inf")
BIG = 1 << 30

B, N, D = 8, 5000, 85
NCLS = D - 5
NPAD = 5120
TILE0 = 1280
TILE3 = 1160
PAD = 1280
CH = 640
NCHUNK = PAD // CH
CG = CH // 16
LAST3 = TILE3 - CH


def _sc_body(pred_hbm, cc_hbm, cp_hbm, mask_hbm, scores_hbm,
             slab, conf_buf, valid_buf, mask_buf, scores_buf,
             cc_buf, cp_buf, v16f, v16i, counts_sh, top5v_sh, top5i_sh):
    c = lax.axis_index("c")
    s = lax.axis_index("s")
    wid = c * 16 + s
    b = wid // 4
    j = wid % 4
    b_loc = s // 4
    nbase = j * TILE0
    n = jnp.where(j == 3, TILE3, TILE0)
    iota16 = lax.iota(jnp.int32, 16)

    def run_chunk(k, cnt):
        def group(g, cnt):
            lr = g * 16 + iota16
            glr = k * CH + lr

            def fld(f):
                return plsc.load_gather(
                    slab, [lr, jnp.full((16,), f, jnp.int32)])

            v0 = fld(0)
            v1 = fld(1)
            v2 = fld(2)
            v3 = fld(3)
            conf_v = fld(4)

            ccv = jnp.zeros((16,), jnp.float32)
            cpv = jnp.zeros((16,), jnp.int32)
            for i in range(16):
                r = g * 16 + i
                va = slab[r, pl.ds(0, 16)]
                vb = slab[r, pl.ds(16, 16)]
                vc = slab[r, pl.ds(32, 16)]
                vd = slab[r, pl.ds(48, 16)]
                ve = slab[r, pl.ds(64, 16)]
                vf = slab[r, pl.ds(69, 16)]
                va = jnp.where(iota16 >= 5, va, NEG_INF)
                m = jnp.maximum(jnp.maximum(jnp.maximum(va, vb),
                                            jnp.maximum(vc, vd)),
                                jnp.maximum(ve, vf))
                mval = jnp.max(m)
                cand = jnp.minimum(
                    jnp.minimum(
                        jnp.minimum(
                            jnp.where(va == mval, iota16 - 5, BIG),
                            jnp.where(vb == mval, iota16 + 11, BIG)),
                        jnp.minimum(
                            jnp.where(vc == mval, iota16 + 27, BIG),
                            jnp.where(vd == mval, iota16 + 43, BIG))),
                    jnp.minimum(
                        jnp.where(ve == mval, iota16 + 59, BIG),
                        jnp.where(vf == mval, iota16 + 64, BIG)))
                cidx = jnp.min(cand)
                here = iota16 == i
                ccv = jnp.where(here, mval, ccv)
                cpv = jnp.where(here, cidx, cpv)

            real = glr < n
            cmaskv = (conf_v >= CONF_THRES) & real
            validv = (v2 > v0) & (v3 > v1)
            finalv = cmaskv & validv
            cnt = cnt + jnp.where(cmaskv, jnp.int32(1), jnp.int32(0))

            sl = pl.ds(k * CH + g * 16, 16)
            mask_buf[sl] = jnp.where(finalv, jnp.int32(1), jnp.int32(0))
            scores_buf[sl] = jnp.where(finalv, conf_v, NEG_INF)
            conf_buf[sl] = jnp.where(real, conf_v, NEG_INF)
            valid_buf[sl] = jnp.where(validv, jnp.int32(1), jnp.int32(0))
            cc_buf[sl] = ccv
            cp_buf[sl] = cpv.astype(jnp.float32)
            return cnt

        return plsc.parallel_loop(0, CG, carry=cnt, unroll=2)(group)

    cnt16 = jnp.zeros((16,), jnp.int32)
    for k in range(NCHUNK):
        row0 = nbase + k * CH
        if k < NCHUNK - 1:
            pltpu.sync_copy(pred_hbm.at[b, pl.ds(row0, CH)], slab)
        else:
            @pl.when(j < 3)
            def _full_last():
                pltpu.sync_copy(pred_hbm.at[b, pl.ds(row0, CH)], slab)

            @pl.when(j == 3)
            def _part_last():
                pltpu.sync_copy(pred_hbm.at[b, pl.ds(row0, LAST3)],
                                slab.at[pl.ds(0, LAST3)])

        cnt16 = run_chunk(k, cnt16)

    obase = b * NPAD + nbase
    pltpu.sync_copy(cc_buf, cc_hbm.at[pl.ds(obase, PAD)])
    pltpu.sync_copy(cp_buf, cp_hbm.at[pl.ds(obase, PAD)])
    pltpu.sync_copy(mask_buf, mask_hbm.at[pl.ds(obase, PAD)])
    pltpu.sync_copy(scores_buf, scores_hbm.at[pl.ds(obase, PAD)])

    total = jnp.sum(cnt16)
    v16i[...] = jnp.where(iota16 == 0, total, jnp.int32(0))
    pltpu.sync_copy(v16i, counts_sh.at[s])
    plsc.subcore_barrier()
    s0 = b_loc * 4
    above = jnp.int32(0)
    for m in range(4):
        pltpu.sync_copy(counts_sh.at[s0 + m], v16i)
        above = above + jnp.sum(v16i[...])
    need_fb = above == 0

    NG = PAD // 16
    v16f[...] = jnp.full((16,), NEG_INF, jnp.float32)
    pltpu.sync_copy(v16f, top5v_sh.at[s])
    v16i[...] = jnp.full((16,), BIG, jnp.int32)
    pltpu.sync_copy(v16i, top5i_sh.at[s])

    @pl.when(need_fb)
    def _local_top5():
        t5v = jnp.full((16,), NEG_INF, jnp.float32)
        t5i = jnp.full((16,), BIG, jnp.int32)
        sel = []
        for k in range(5):
            m16 = lax.fori_loop(
                0, NG,
                lambda g, m: jnp.maximum(m, conf_buf[pl.ds(g * 16, 16)]),
                jnp.full((16,), NEG_INF, jnp.float32))
            mval = jnp.max(m16)

            def first_idx(g, cur):
                v = conf_buf[pl.ds(g * 16, 16)]
                cand = jnp.min(jnp.where(v == mval, g * 16 + iota16, BIG))
                return jnp.minimum(cur, cand)

            idx = lax.fori_loop(0, NG, first_idx, jnp.int32(BIG))
            t5v = jnp.where(iota16 == k, mval, t5v)
            t5i = jnp.where(iota16 == k, nbase + idx, t5i)
            sel.append((idx, mval))
            gsel = idx // 16
            hole = pl.ds(gsel * 16, 16)
            conf_buf[hole] = jnp.where(gsel * 16 + iota16 == idx, NEG_INF,
                                       conf_buf[hole])
        for idx, mval in reversed(sel):
            gsel = idx // 16
            hole = pl.ds(gsel * 16, 16)
            conf_buf[hole] = jnp.where(gsel * 16 + iota16 == idx, mval,
                                       conf_buf[hole])
        v16f[...] = t5v
        pltpu.sync_copy(v16f, top5v_sh.at[s])
        v16i[...] = t5i
        pltpu.sync_copy(v16i, top5i_sh.at[s])

    plsc.subcore_barrier()

    @pl.when(need_fb)
    def _apply_fb():
        vs, ix = [], []
        for m in range(4):
            pltpu.sync_copy(top5v_sh.at[s0 + m], v16f)
            vs.append(v16f[...])
            pltpu.sync_copy(top5i_sh.at[s0 + m], v16i)
            ix.append(v16i[...])
        sel = []
        for k in range(5):
            mval = jnp.max(jnp.maximum(jnp.maximum(vs[0], vs[1]),
                                       jnp.maximum(vs[2], vs[3])))
            cand = jnp.int32(BIG)
            for m in range(4):
                cand = jnp.minimum(
                    cand, jnp.min(jnp.where(vs[m] == mval, ix[m], BIG)))
            for m in range(4):
                hit = (vs[m] == mval) & (ix[m] == cand)
                vs[m] = jnp.where(hit, NEG_INF, vs[m])
            sel.append(cand)

        def rebuild(g, _):
            lr = g * 16 + iota16
            blr = nbase + lr
            inb = ((blr == sel[0]) | (blr == sel[1]) | (blr == sel[2])
                   | (blr == sel[3]) | (blr == sel[4]))
            sl = pl.ds(g * 16, 16)
            fin = inb & (valid_buf[sl] != 0) & (lr < n)
            mask_buf[sl] = jnp.where(fin, jnp.int32(1), jnp.int32(0))
            scores_buf[sl] = jnp.where(fin, conf_buf[sl], NEG_INF)
            return 0

        lax.fori_loop(0, NG, rebuild, 0)
        pltpu.sync_copy(mask_buf, mask_hbm.at[pl.ds(obase, PAD)])
        pltpu.sync_copy(scores_buf, scores_hbm.at[pl.ds(obase, PAD)])


_sc_call = pl.kernel(
    _sc_body,
    out_type=(
        jax.ShapeDtypeStruct((B * NPAD,), jnp.float32),
        jax.ShapeDtypeStruct((B * NPAD,), jnp.float32),
        jax.ShapeDtypeStruct((B * NPAD,), jnp.int32),
        jax.ShapeDtypeStruct((B * NPAD,), jnp.float32),
    ),
    mesh=plsc.VectorSubcoreMesh(core_axis_name="c", subcore_axis_name="s",
                                num_cores=2, num_subcores=16),
    compiler_params=pltpu.CompilerParams(needs_layout_passes=False),
    scratch_types=[
        pltpu.VMEM((CH, D), jnp.float32),
        pltpu.VMEM((PAD,), jnp.float32),
        pltpu.VMEM((PAD,), jnp.int32),
        pltpu.VMEM((PAD,), jnp.int32),
        pltpu.VMEM((PAD,), jnp.float32),
        pltpu.VMEM((PAD,), jnp.float32),
        pltpu.VMEM((PAD,), jnp.float32),
        pltpu.VMEM((16,), jnp.float32),
        pltpu.VMEM((16,), jnp.int32),
        pltpu.VMEM_SHARED((16, 16), jnp.int32),
        pltpu.VMEM_SHARED((16, 16), jnp.float32),
        pltpu.VMEM_SHARED((16, 16), jnp.int32),
    ],
)


@jax.jit
def kernel(prediction):
    cc, cp, mask_i32, scores = _sc_call(prediction)
    cc = cc.reshape(B, NPAD)[:, :N, None]
    cp = cp.reshape(B, NPAD)[:, :N, None]
    det = jnp.concatenate([prediction[:, :, :5], cc, cp], axis=-1)
    mask = mask_i32.reshape(B, NPAD)[:, :N] != 0
    return det, mask, scores.reshape(B, NPAD)[:, :N]
